# Initial kernel scaffold; baseline (speedup 1.0000x reference)
import jax
import jax.numpy as jnp
from jax.experimental import pallas as pl


def kernel(inputs):
    raise NotImplementedError("write your pallas kernel here")


# speedup vs baseline: 11.8452x; 11.8452x over previous
import jax
import jax.numpy as jnp
from jax.experimental import pallas as pl

_NUM_CLASSES = 80
_SCORE_THRESHOLD = 0.05
_IOU_THRESHOLD = 0.5
_MAX_BOXES = 10
_N = 64 * 64 * 5
_ROWS = _N // 128
_AW = (1.08, 3.42, 6.63, 9.42, 16.62)
_AH = (1.19, 4.41, 11.38, 5.11, 10.52)


def _postproc_kernel(xt_ref, out_ref):
    f = (jax.lax.broadcasted_iota(jnp.int32, (_ROWS, 128), 0) * 128
         + jax.lax.broadcasted_iota(jnp.int32, (_ROWS, 128), 1))
    a = f % 5
    cell = f // 5
    gx = (cell % 64).astype(jnp.float32)
    gy = (cell // 64).astype(jnp.float32)

    aw = jnp.full(f.shape, _AW[4], jnp.float32)
    ah = jnp.full(f.shape, _AH[4], jnp.float32)
    for k in range(3, -1, -1):
        aw = jnp.where(a == k, _AW[k], aw)
        ah = jnp.where(a == k, _AH[k], ah)

    tx = xt_ref[0]
    ty = xt_ref[1]
    tw = xt_ref[2]
    th = xt_ref[3]
    tc = xt_ref[4]

    inv = 1.0 / 64.0
    bx = (jax.nn.sigmoid(tx) + gx) * inv
    by = (jax.nn.sigmoid(ty) + gy) * inv
    bw = jnp.exp(tw) * aw * inv
    bh = jnp.exp(th) * ah * inv
    conf = jax.nn.sigmoid(tc)

    y1 = by - bh * 0.5
    x1 = bx - bw * 0.5
    y2 = by + bh * 0.5
    x2 = bx + bw * 0.5

    m = xt_ref[5]
    for c in range(1, _NUM_CLASSES):
        m = jnp.maximum(m, xt_ref[5 + c])
    ssum = jnp.zeros(f.shape, jnp.float32)
    cls = jnp.full(f.shape, _NUM_CLASSES, jnp.int32)
    for c in range(_NUM_CLASSES - 1, -1, -1):
        logit = xt_ref[5 + c]
        ssum = ssum + jnp.exp(logit - m)
        cls = jnp.where(logit >= m, c, cls)
    clsf = cls.astype(jnp.float32)

    score = conf / ssum
    smask = jnp.where(score >= _SCORE_THRESHOLD, score, -1.0)

    areas = (y2 - y1) * (x2 - x1)
    li = jax.lax.broadcasted_iota(jnp.int32, (1, 128), 1)
    s = smask
    rows = []
    for _ in range(_MAX_BOXES):
        mx = jnp.max(s)
        idx = jnp.min(jnp.where(s >= mx, f, _N))
        sel = f == idx
        selm = sel.astype(jnp.float32)
        by1 = jnp.sum(y1 * selm)
        bx1 = jnp.sum(x1 * selm)
        by2 = jnp.sum(y2 * selm)
        bx2 = jnp.sum(x2 * selm)
        bsc = jnp.sum(smask * selm)
        bcl = jnp.sum(clsf * selm)
        barea = (by2 - by1) * (bx2 - bx1)
        inter = (jnp.maximum(jnp.minimum(by2, y2) - jnp.maximum(by1, y1), 0.0)
                 * jnp.maximum(jnp.minimum(bx2, x2) - jnp.maximum(bx1, x1), 0.0))
        iou = inter / (barea + areas - inter + 1e-9)
        s = jnp.where((iou > _IOU_THRESHOLD) | sel, -1.0, s)
        valid = jnp.where(bsc >= _SCORE_THRESHOLD, 1.0, 0.0)
        row = jnp.zeros((1, 128), jnp.float32)
        for j, v in enumerate((by1, bx1, by2, bx2, bsc, bcl)):
            row = jnp.where(li == j, v * valid, row)
        rows.append(row)
    out_ref[...] = jnp.concatenate(rows, axis=0)


@jax.jit
def kernel(inputs):
    xt = (inputs.reshape(_N, 5 + _NUM_CLASSES)
          .T.reshape(5 + _NUM_CLASSES, _ROWS, 128))
    out = pl.pallas_call(
        _postproc_kernel,
        out_shape=jax.ShapeDtypeStruct((_MAX_BOXES, 128), jnp.float32),
    )(xt)
    return out[:, :6]
